# baseline (speedup 1.0000x reference)
import jax
import jax.numpy as jnp
from jax import lax
from jax.experimental import pallas as pl
from jax.experimental.pallas import tpu as pltpu
from jax.experimental.pallas import tpu_sc as plsc

_N = 100000
_E = 1600000
_NC, _NS = 2, 16
_HALF = _N // 2
_HALFP = 50176
_OWN = _HALFP // _NS
_ACC_ROWS = _HALFP
_SB = 1024
_CHK = 128
_EPAD = 1605632
_EPT = _EPAD // _NS
_NSB = _EPT // _SB
_KD = 256
_CHD = _KD // 128
_EPW = _EPAD // (_NC * _NS)
_NB_DEG = _EPW // _KD
_NPAD = 100352


def _mesh():
    return plsc.VectorSubcoreMesh(
        core_axis_name="c", subcore_axis_name="s",
        num_cores=_NC, num_subcores=_NS)


def _deg_body(dst_hbm, ew_hbm, out0_hbm, out1_hbm, idx_v, ew_v, zb_v, acc_sh):
    c = lax.axis_index("c")
    s = lax.axis_index("s")
    w = c * _NS + s

    def _z(i, _):
        zb_v[pl.ds(i * 16, 16)] = jnp.zeros((16,), jnp.float32)
        return 0
    lax.fori_loop(0, _KD // 16, _z, 0)

    npt = _NPAD // _NS
    for i in range(npt // _KD):
        pltpu.sync_copy(zb_v.at[pl.ds(0, _KD)],
                        acc_sh.at[pl.ds(s * npt + i * _KD, _KD)])
    rem = npt % _KD
    if rem:
        pltpu.sync_copy(zb_v.at[pl.ds(0, rem)],
                        acc_sh.at[pl.ds(s * npt + (npt // _KD) * _KD, rem)])
    plsc.subcore_barrier()

    row0 = w * (_EPW // 128)

    def _blk(b, _):
        rb = row0 + b * _CHD
        pltpu.sync_copy(dst_hbm.at[pl.ds(rb, _CHD)], idx_v)
        pltpu.sync_copy(ew_hbm.at[pl.ds(rb, _CHD)], ew_v)
        for ch in range(_CHD):
            pltpu.sync_copy(ew_v.at[ch], acc_sh.at[idx_v.at[ch]], add=True)
        return 0
    lax.fori_loop(0, _NB_DEG, _blk, 0)
    plsc.subcore_barrier()

    for out_hbm, cc in ((out0_hbm, 0), (out1_hbm, 1)):
        @pl.when(c == cc)
        def _():
            for i in range(npt // _KD):
                pltpu.sync_copy(acc_sh.at[pl.ds(s * npt + i * _KD, _KD)],
                                zb_v.at[pl.ds(0, _KD)])
                pltpu.sync_copy(zb_v.at[pl.ds(0, _KD)],
                                out_hbm.at[pl.ds(s * npt + i * _KD, _KD)])
            if rem:
                o = s * npt + (npt // _KD) * _KD
                pltpu.sync_copy(acc_sh.at[pl.ds(o, rem)],
                                zb_v.at[pl.ds(0, rem)])
                pltpu.sync_copy(zb_v.at[pl.ds(0, rem)],
                                out_hbm.at[pl.ds(o, rem)])


def _deg_call(dst_p, ew_p):
    fn = pl.kernel(
        _deg_body,
        out_type=(jax.ShapeDtypeStruct((_NPAD,), jnp.float32),
                  jax.ShapeDtypeStruct((_NPAD,), jnp.float32)),
        mesh=_mesh(),
        compiler_params=pltpu.CompilerParams(use_tc_tiling_on_sc=False),
        scratch_types=[
            pltpu.VMEM((_CHD, 128), jnp.int32),
            pltpu.VMEM((_CHD, 128), jnp.float32),
            pltpu.VMEM((_KD,), jnp.float32),
            pltpu.VMEM_SHARED((_NPAD,), jnp.float32),
        ],
    )
    return fn(dst_p, ew_p)


def _make_edge_pass(nf):
    def body(src_hbm, dst_hbm, ew_hbm, *rest):
        h_hbms = rest[:nf]
        out_hbms = rest[nf:2 * nf]
        (isA, idA, ewA, isB, idB, ewB, r0, r1, bnc, acc_sh,
         slA, slB, sg0, sg1, ss0, ss1) = rest[2 * nf:]
        stg = ((isA, idA, ewA, slA), (isB, idB, ewB, slB))
        rows = (r0, r1)
        sg = (sg0, sg1)
        ss = (ss0, ss1)
        c = lax.axis_index("c")
        s = lax.axis_index("s")
        base_node = c * _HALF
        row0 = s * (_EPT // 128)
        ebase = s * _EPT
        z16f = jnp.zeros((16,), jnp.float32)

        def _fire_lin(sb, st):
            rb = row0 + sb * (_SB // 128)
            pltpu.async_copy(src_hbm.at[pl.ds(rb, _SB // 128)], st[0], st[3])
            pltpu.async_copy(dst_hbm.at[pl.ds(rb, _SB // 128)], st[1], st[3])
            pltpu.async_copy(ew_hbm.at[pl.ds(ebase + sb * _SB, _SB)], st[2],
                             st[3])

        def _wait_lin(sb, st):
            rb = row0 + sb * (_SB // 128)
            pltpu.make_async_copy(src_hbm.at[pl.ds(rb, _SB // 128)], st[0],
                                  st[3]).wait()
            pltpu.make_async_copy(dst_hbm.at[pl.ds(rb, _SB // 128)], st[1],
                                  st[3]).wait()
            pltpu.make_async_copy(ew_hbm.at[pl.ds(ebase + sb * _SB, _SB)],
                                  st[2], st[3]).wait()

        def _fire_gather(h_hbm, st, ch, rset):
            pltpu.async_copy(h_hbm.at[st[0].at[ch]], rows[rset], sg[rset])

        def _wait_gather(h_hbm, st, ch, rset):
            pltpu.make_async_copy(h_hbm.at[st[0].at[ch]], rows[rset],
                                  sg[rset]).wait()

        def _fire_scatter(st, ch, rset):
            pltpu.async_copy(rows[rset], acc_sh.at[st[1].at[ch]],
                             ss[rset], add=True)

        def _wait_scatter(st, ch, rset):
            pltpu.make_async_copy(rows[rset], acc_sh.at[st[1].at[ch]],
                                  ss[rset]).wait()

        def _compute(st, ch, rset):
            id_v, ew_v, rows_v = st[1], st[2], rows[rset]

            def _ix(v, _):
                lb = v * 16
                d = id_v[ch, pl.ds(lb, 16)]
                e = ew_v[pl.ds(ch * _CHK + lb, 16)]
                loc = d - base_node
                ok = (loc >= 0) & (loc < _HALF)
                id_v[ch, pl.ds(lb, 16)] = jnp.clip(loc, 0, _HALF - 1)
                ew_v[pl.ds(ch * _CHK + lb, 16)] = jnp.where(ok, e, 0.0)
                return 0
            lax.fori_loop(0, _CHK // 16, _ix, 0)

            def _sc(g, _):
                e_vec = ew_v[pl.ds(ch * _CHK + g * 16, 16)]
                for l in range(16):
                    jj = g * 16 + l
                    e = e_vec[l]
                    rows_v[jj, pl.ds(0, 16)] = rows_v[jj, pl.ds(0, 16)] * e
                    rows_v[jj, pl.ds(16, 16)] = rows_v[jj, pl.ds(16, 16)] * e
                return 0
            lax.fori_loop(0, _CHK // 16, _sc, 0)

        for r in range(nf):
            def _zr(j, _):
                bnc[j, pl.ds(0, 16)] = z16f
                bnc[j, pl.ds(16, 16)] = z16f
                return 0
            lax.fori_loop(0, 128, _zr, 0)
            plsc.subcore_barrier()
            for i in range(_OWN // 128):
                pltpu.sync_copy(bnc,
                                acc_sh.at[pl.ds(s * _OWN + i * 128, 128)])
            arem = _OWN % 128
            if arem:
                pltpu.sync_copy(
                    bnc.at[pl.ds(0, arem)],
                    acc_sh.at[pl.ds(s * _OWN + (_OWN // 128) * 128, arem)])
            plsc.subcore_barrier()

            h_hbm = h_hbms[r]

            _fire_lin(0, stg[0])
            _fire_lin(1, stg[1])

            def _iter(i, _):
                _wait_lin(2 * i, stg[0])
                _wait_lin(2 * i + 1, stg[1])
                _fire_gather(h_hbm, stg[0], 0, 0)
                for t in range(16):
                    st = stg[0] if t < 8 else stg[1]
                    ch = t % 8
                    rset = t % 2
                    _wait_gather(h_hbm, st, ch, rset)
                    if t < 15:
                        nst = stg[0] if t + 1 < 8 else stg[1]
                        if t >= 1:
                            pst = stg[0] if t - 1 < 8 else stg[1]
                            _wait_scatter(pst, (t - 1) % 8, (t + 1) % 2)
                        _fire_gather(h_hbm, nst, (t + 1) % 8, (t + 1) % 2)
                    _compute(st, ch, rset)
                    _fire_scatter(st, ch, rset)
                    if t == 8:
                        @pl.when(i < _NSB // 2 - 1)
                        def _():
                            _fire_lin(2 * i + 2, stg[0])
                _wait_scatter(stg[1], 6, 0)
                _wait_scatter(stg[1], 7, 1)

                @pl.when(i < _NSB // 2 - 1)
                def _():
                    _fire_lin(2 * i + 3, stg[1])
                return 0
            lax.fori_loop(0, _NSB // 2, _iter, 0)
            plsc.subcore_barrier()

            out_hbm = out_hbms[r]
            off = 0
            for sz in (128,) * (_OWN // 128) + ((_OWN % 128,)
                                                if _OWN % 128 else ()):
                pltpu.sync_copy(acc_sh.at[pl.ds(s * _OWN + off, sz)],
                                bnc.at[pl.ds(0, sz)])
                pltpu.sync_copy(
                    bnc.at[pl.ds(0, sz)],
                    out_hbm.at[pl.ds(c * _HALFP + s * _OWN + off, sz)])
                off += sz

    out_type = tuple(
        jax.ShapeDtypeStruct((2 * _HALFP, 32), jnp.float32)
        for _ in range(nf))
    return pl.kernel(
        body,
        out_type=out_type,
        mesh=_mesh(),
        compiler_params=pltpu.CompilerParams(use_tc_tiling_on_sc=False),
        scratch_types=[
            pltpu.VMEM((_SB // 128, 128), jnp.int32),
            pltpu.VMEM((_SB // 128, 128), jnp.int32),
            pltpu.VMEM((_SB,), jnp.float32),
            pltpu.VMEM((_SB // 128, 128), jnp.int32),
            pltpu.VMEM((_SB // 128, 128), jnp.int32),
            pltpu.VMEM((_SB,), jnp.float32),
            pltpu.VMEM((128, 32), jnp.float32),
            pltpu.VMEM((128, 32), jnp.float32),
            pltpu.VMEM((128, 32), jnp.float32),
            pltpu.VMEM_SHARED((_ACC_ROWS, 32), jnp.float32),
            pltpu.SemaphoreType.DMA,
            pltpu.SemaphoreType.DMA,
            pltpu.SemaphoreType.DMA,
            pltpu.SemaphoreType.DMA,
            pltpu.SemaphoreType.DMA,
            pltpu.SemaphoreType.DMA,
        ],
    )


_R = 1000
_G = _N // _R


def _row_call(body, out_dims, ins, full_mask):
    in_specs = []
    for a, full in zip(ins, full_mask):
        if full:
            in_specs.append(
                pl.BlockSpec(a.shape, lambda i, nd=a.ndim: (0,) * nd))
        else:
            in_specs.append(
                pl.BlockSpec((_R, a.shape[1]), lambda i: (i, 0)))
    out_specs = [pl.BlockSpec((_R, d), lambda i: (i, 0)) for d in out_dims]
    out_shape = [jax.ShapeDtypeStruct((_N, d), jnp.float32) for d in out_dims]
    return pl.pallas_call(
        body, grid=(_G,), in_specs=in_specs,
        out_specs=out_specs, out_shape=out_shape)(*ins)


def _prep_body(st, nf, d0, d1, w1, ht_o, h_o, dinv_o):
    deg = d0[...] + d1[...] + 1.0
    dinv = jnp.where(deg > 0, lax.rsqrt(jnp.maximum(deg, 1e-12)), 0.0)
    h = (jnp.dot(nf[...], w1[0:5, :], preferred_element_type=jnp.float32)
         + st[...] * w1[5:6, :])
    h_o[...] = h
    ht_o[...] = dinv * h
    dinv_o[...] = dinv


def _d1_body(s1, h1, dinv_r, b, w, h_o, htlo_o, hthi_o):
    dinv = dinv_r[...]
    y = dinv * s1[...] + (dinv * dinv) * h1[...] + b[...]
    h = jnp.dot(y, w[...], preferred_element_type=jnp.float32)
    h_o[...] = h
    ht = dinv * h
    htlo_o[...] = ht[:, 0:32]
    hthi_o[...] = ht[:, 32:64]


def _d2_body(slo, shi, h2, dinv_r, b, w, h_o, htlo_o, hthi_o):
    dinv = dinv_r[...]
    h2v = h2[...]
    bv = b[...]
    ylo = dinv * slo[...] + (dinv * dinv) * h2v[:, 0:32] + bv[:, 0:32]
    yhi = dinv * shi[...] + (dinv * dinv) * h2v[:, 32:64] + bv[:, 32:64]
    h = (jnp.dot(ylo, w[0:32, :], preferred_element_type=jnp.float32)
         + jnp.dot(yhi, w[32:64, :], preferred_element_type=jnp.float32))
    h_o[...] = h
    ht = dinv * h
    htlo_o[...] = ht[:, 0:32]
    hthi_o[...] = ht[:, 32:64]


def _d3_body(slo, shi, h3, dinv_r, b, wp1, bp1, wp2, bp2, out):
    dinv = dinv_r[...]
    h3v = h3[...]
    bv = b[...]
    ylo = dinv * slo[...] + (dinv * dinv) * h3v[:, 0:32] + bv[:, 0:32]
    yhi = dinv * shi[...] + (dinv * dinv) * h3v[:, 32:64] + bv[:, 32:64]
    t = jnp.maximum(
        jnp.dot(ylo, wp1[0:32, :], preferred_element_type=jnp.float32)
        + jnp.dot(yhi, wp1[32:64, :], preferred_element_type=jnp.float32)
        + bp1[...], 0.0)
    z = jnp.dot(t, wp2[...], preferred_element_type=jnp.float32) + bp2[...]
    out[...] = jax.nn.sigmoid(z)


def kernel(states, env, node_features, edge_index, edge_attr,
           W1, b1, W2, b2, W3, b3, Wp1, bp1, Wp2, bp2):
    del env
    src = edge_index[0]
    dst = edge_index[1]
    pad = _EPAD - _E
    fill = (jnp.arange(pad, dtype=jnp.int32) * 797) % jnp.int32(_N)
    src_p = jnp.concatenate([src, fill]).reshape(_EPAD // 128, 128)
    dst_p = jnp.concatenate([dst, fill]).reshape(_EPAD // 128, 128)
    ew_flat = jnp.concatenate([edge_attr, jnp.zeros((pad,), jnp.float32)])
    ew_p = ew_flat.reshape(_EPAD // 128, 128)

    degp0, degp1 = _deg_call(dst_p, ew_p)
    d0 = degp0[:_N].reshape(_N, 1)
    d1 = degp1[:_N].reshape(_N, 1)

    def _unpad(o):
        return jnp.concatenate([o[:_HALF], o[_HALFP:_HALFP + _HALF]])

    st = states.reshape(_N, 1)
    ht1, h1, dinv = _row_call(
        _prep_body, (32, 32, 1),
        (st, node_features, d0, d1, W1),
        (False, False, False, False, True))

    edge32 = _make_edge_pass(1)
    edge64 = _make_edge_pass(2)

    (s1,) = edge32(src_p, dst_p, ew_flat, ht1)
    s1 = _unpad(s1)
    h2, ht2lo, ht2hi = _row_call(
        _d1_body, (64, 32, 32),
        (s1, h1, dinv, b1.reshape(1, 32), W2),
        (False, False, False, True, True))

    s2lo, s2hi = edge64(src_p, dst_p, ew_flat, ht2lo, ht2hi)
    s2lo, s2hi = _unpad(s2lo), _unpad(s2hi)
    h3, ht3lo, ht3hi = _row_call(
        _d2_body, (64, 32, 32),
        (s2lo, s2hi, h2, dinv, b2.reshape(1, 64), W3),
        (False, False, False, False, True, True))

    s3lo, s3hi = edge64(src_p, dst_p, ew_flat, ht3lo, ht3hi)
    s3lo, s3hi = _unpad(s3lo), _unpad(s3hi)
    (preds,) = _row_call(
        _d3_body, (1,),
        (s3lo, s3hi, h3, dinv, b3.reshape(1, 64),
         Wp1, bp1.reshape(1, 32), Wp2, bp2.reshape(1, 1)),
        (False, False, False, False, True, True, True, True, True))
    return preds.reshape(-1)


# speedup vs baseline: 1.2990x; 1.2990x over previous
import jax
import jax.numpy as jnp
from jax import lax
from jax.experimental import pallas as pl
from jax.experimental.pallas import tpu as pltpu
from jax.experimental.pallas import tpu_sc as plsc

_N = 100000
_E = 1600000
_NC, _NS = 2, 16
_HALF = _N // 2
_HALFP = 50176
_OWN = _HALFP // _NS
_ACC_ROWS = _HALFP
_SB = 1024
_CHK = 128
_EPAD = 1605632
_EPT = _EPAD // _NS
_NSB = _EPT // _SB
_KD = 256
_CHD = _KD // 128
_EPW = _EPAD // (_NC * _NS)
_NB_DEG = _EPW // _KD
_NPAD = 100352


def _mesh():
    return plsc.VectorSubcoreMesh(
        core_axis_name="c", subcore_axis_name="s",
        num_cores=_NC, num_subcores=_NS)


def _deg_body(dst_hbm, ew_hbm, out0_hbm, out1_hbm, idx_v, ew_v, zb_v, acc_sh):
    c = lax.axis_index("c")
    s = lax.axis_index("s")
    w = c * _NS + s

    def _z(i, _):
        zb_v[pl.ds(i * 16, 16)] = jnp.zeros((16,), jnp.float32)
        return 0
    lax.fori_loop(0, _KD // 16, _z, 0)

    npt = _NPAD // _NS
    for i in range(npt // _KD):
        pltpu.sync_copy(zb_v.at[pl.ds(0, _KD)],
                        acc_sh.at[pl.ds(s * npt + i * _KD, _KD)])
    rem = npt % _KD
    if rem:
        pltpu.sync_copy(zb_v.at[pl.ds(0, rem)],
                        acc_sh.at[pl.ds(s * npt + (npt // _KD) * _KD, rem)])
    plsc.subcore_barrier()

    row0 = w * (_EPW // 128)

    def _blk(b, _):
        rb = row0 + b * _CHD
        pltpu.sync_copy(dst_hbm.at[pl.ds(rb, _CHD)], idx_v)
        pltpu.sync_copy(ew_hbm.at[pl.ds(rb, _CHD)], ew_v)
        for ch in range(_CHD):
            pltpu.sync_copy(ew_v.at[ch], acc_sh.at[idx_v.at[ch]], add=True)
        return 0
    lax.fori_loop(0, _NB_DEG, _blk, 0)
    plsc.subcore_barrier()

    for out_hbm, cc in ((out0_hbm, 0), (out1_hbm, 1)):
        @pl.when(c == cc)
        def _():
            for i in range(npt // _KD):
                pltpu.sync_copy(acc_sh.at[pl.ds(s * npt + i * _KD, _KD)],
                                zb_v.at[pl.ds(0, _KD)])
                pltpu.sync_copy(zb_v.at[pl.ds(0, _KD)],
                                out_hbm.at[pl.ds(s * npt + i * _KD, _KD)])
            if rem:
                o = s * npt + (npt // _KD) * _KD
                pltpu.sync_copy(acc_sh.at[pl.ds(o, rem)],
                                zb_v.at[pl.ds(0, rem)])
                pltpu.sync_copy(zb_v.at[pl.ds(0, rem)],
                                out_hbm.at[pl.ds(o, rem)])


def _deg_call(dst_p, ew_p):
    fn = pl.kernel(
        _deg_body,
        out_type=(jax.ShapeDtypeStruct((_NPAD,), jnp.float32),
                  jax.ShapeDtypeStruct((_NPAD,), jnp.float32)),
        mesh=_mesh(),
        compiler_params=pltpu.CompilerParams(use_tc_tiling_on_sc=False),
        scratch_types=[
            pltpu.VMEM((_CHD, 128), jnp.int32),
            pltpu.VMEM((_CHD, 128), jnp.float32),
            pltpu.VMEM((_KD,), jnp.float32),
            pltpu.VMEM_SHARED((_NPAD,), jnp.float32),
        ],
    )
    return fn(dst_p, ew_p)


def _make_edge_pass(nf):
    def body(src_hbm, dst_hbm, ew_hbm, *rest):
        h_hbms = rest[:nf]
        out_hbms = rest[nf:2 * nf]
        (isA, idA, ewA, isB, idB, ewB, r0, r1, r2, r3, bnc, acc_sh,
         slA, slB, sg0, sg1, sg2, sg3, ss0, ss1, ss2, ss3) = rest[2 * nf:]
        stg = ((isA, idA, ewA, slA), (isB, idB, ewB, slB))
        rows = (r0, r1, r2, r3)
        sg = (sg0, sg1, sg2, sg3)
        ss = (ss0, ss1, ss2, ss3)
        c = lax.axis_index("c")
        s = lax.axis_index("s")
        base_node = c * _HALF
        row0 = s * (_EPT // 128)
        ebase = s * _EPT
        z16f = jnp.zeros((16,), jnp.float32)

        def _fire_lin(sb, st):
            rb = row0 + sb * (_SB // 128)
            pltpu.async_copy(src_hbm.at[pl.ds(rb, _SB // 128)], st[0], st[3])
            pltpu.async_copy(dst_hbm.at[pl.ds(rb, _SB // 128)], st[1], st[3])
            pltpu.async_copy(ew_hbm.at[pl.ds(ebase + sb * _SB, _SB)], st[2],
                             st[3])

        def _wait_lin(sb, st):
            rb = row0 + sb * (_SB // 128)
            pltpu.make_async_copy(src_hbm.at[pl.ds(rb, _SB // 128)], st[0],
                                  st[3]).wait()
            pltpu.make_async_copy(dst_hbm.at[pl.ds(rb, _SB // 128)], st[1],
                                  st[3]).wait()
            pltpu.make_async_copy(ew_hbm.at[pl.ds(ebase + sb * _SB, _SB)],
                                  st[2], st[3]).wait()

        def _fire_gather(h_hbm, st, ch, rset):
            pltpu.async_copy(h_hbm.at[st[0].at[ch]], rows[rset], sg[rset])

        def _wait_gather(h_hbm, st, ch, rset):
            pltpu.make_async_copy(h_hbm.at[st[0].at[ch]], rows[rset],
                                  sg[rset]).wait()

        def _fire_scatter(st, ch, rset):
            pltpu.async_copy(rows[rset], acc_sh.at[st[1].at[ch]],
                             ss[rset], add=True)

        def _wait_scatter(st, ch, rset):
            pltpu.make_async_copy(rows[rset], acc_sh.at[st[1].at[ch]],
                                  ss[rset]).wait()

        def _compute(st, ch, rset):
            id_v, ew_v, rows_v = st[1], st[2], rows[rset]

            def _ix(v, _):
                lb = v * 16
                d = id_v[ch, pl.ds(lb, 16)]
                e = ew_v[pl.ds(ch * _CHK + lb, 16)]
                loc = d - base_node
                ok = (loc >= 0) & (loc < _HALF)
                spread = jnp.bitwise_and(d, 16383)
                id_v[ch, pl.ds(lb, 16)] = jnp.where(ok, loc, spread)
                ew_v[pl.ds(ch * _CHK + lb, 16)] = jnp.where(ok, e, 0.0)
                return 0
            lax.fori_loop(0, _CHK // 16, _ix, 0)

            def _sc(g, _):
                e_vec = ew_v[pl.ds(ch * _CHK + g * 16, 16)]
                for l in range(16):
                    jj = g * 16 + l
                    e = e_vec[l]
                    rows_v[jj, pl.ds(0, 16)] = rows_v[jj, pl.ds(0, 16)] * e
                    rows_v[jj, pl.ds(16, 16)] = rows_v[jj, pl.ds(16, 16)] * e
                return 0
            lax.fori_loop(0, _CHK // 16, _sc, 0)

        for r in range(nf):
            def _zr(j, _):
                bnc[j, pl.ds(0, 16)] = z16f
                bnc[j, pl.ds(16, 16)] = z16f
                return 0
            lax.fori_loop(0, 128, _zr, 0)
            plsc.subcore_barrier()
            for i in range(_OWN // 128):
                pltpu.sync_copy(bnc,
                                acc_sh.at[pl.ds(s * _OWN + i * 128, 128)])
            arem = _OWN % 128
            if arem:
                pltpu.sync_copy(
                    bnc.at[pl.ds(0, arem)],
                    acc_sh.at[pl.ds(s * _OWN + (_OWN // 128) * 128, arem)])
            plsc.subcore_barrier()

            h_hbm = h_hbms[r]

            _fire_lin(0, stg[0])
            _fire_lin(1, stg[1])

            def _iter(i, _):
                _wait_lin(2 * i, stg[0])
                _wait_lin(2 * i + 1, stg[1])
                _fire_gather(h_hbm, stg[0], 0, 0)
                for t in range(16):
                    st = stg[0] if t < 8 else stg[1]
                    ch = t % 8
                    rset = t % 4
                    _wait_gather(h_hbm, st, ch, rset)
                    if t < 15:
                        nst = stg[0] if t + 1 < 8 else stg[1]
                        if t >= 3:
                            pst = stg[0] if t - 3 < 8 else stg[1]
                            _wait_scatter(pst, (t - 3) % 8, (t + 1) % 4)
                        _fire_gather(h_hbm, nst, (t + 1) % 8, (t + 1) % 4)
                    _compute(st, ch, rset)
                    _fire_scatter(st, ch, rset)
                    if t == 10:
                        @pl.when(i < _NSB // 2 - 1)
                        def _():
                            _fire_lin(2 * i + 2, stg[0])
                _wait_scatter(stg[1], 4, 0)
                _wait_scatter(stg[1], 5, 1)
                _wait_scatter(stg[1], 6, 2)
                _wait_scatter(stg[1], 7, 3)

                @pl.when(i < _NSB // 2 - 1)
                def _():
                    _fire_lin(2 * i + 3, stg[1])
                return 0
            lax.fori_loop(0, _NSB // 2, _iter, 0)
            plsc.subcore_barrier()

            out_hbm = out_hbms[r]
            off = 0
            for sz in (128,) * (_OWN // 128) + ((_OWN % 128,)
                                                if _OWN % 128 else ()):
                pltpu.sync_copy(acc_sh.at[pl.ds(s * _OWN + off, sz)],
                                bnc.at[pl.ds(0, sz)])
                pltpu.sync_copy(
                    bnc.at[pl.ds(0, sz)],
                    out_hbm.at[pl.ds(c * _HALFP + s * _OWN + off, sz)])
                off += sz

    out_type = tuple(
        jax.ShapeDtypeStruct((2 * _HALFP, 32), jnp.float32)
        for _ in range(nf))
    return pl.kernel(
        body,
        out_type=out_type,
        mesh=_mesh(),
        compiler_params=pltpu.CompilerParams(use_tc_tiling_on_sc=False),
        scratch_types=[
            pltpu.VMEM((_SB // 128, 128), jnp.int32),
            pltpu.VMEM((_SB // 128, 128), jnp.int32),
            pltpu.VMEM((_SB,), jnp.float32),
            pltpu.VMEM((_SB // 128, 128), jnp.int32),
            pltpu.VMEM((_SB // 128, 128), jnp.int32),
            pltpu.VMEM((_SB,), jnp.float32),
            pltpu.VMEM((128, 32), jnp.float32),
            pltpu.VMEM((128, 32), jnp.float32),
            pltpu.VMEM((128, 32), jnp.float32),
            pltpu.VMEM((128, 32), jnp.float32),
            pltpu.VMEM((128, 32), jnp.float32),
            pltpu.VMEM_SHARED((_ACC_ROWS, 32), jnp.float32),
            pltpu.SemaphoreType.DMA,
            pltpu.SemaphoreType.DMA,
            pltpu.SemaphoreType.DMA,
            pltpu.SemaphoreType.DMA,
            pltpu.SemaphoreType.DMA,
            pltpu.SemaphoreType.DMA,
            pltpu.SemaphoreType.DMA,
            pltpu.SemaphoreType.DMA,
            pltpu.SemaphoreType.DMA,
            pltpu.SemaphoreType.DMA,
        ],
    )


_R = 1000
_G = _N // _R


def _row_call(body, out_dims, ins, full_mask):
    in_specs = []
    for a, full in zip(ins, full_mask):
        if full:
            in_specs.append(
                pl.BlockSpec(a.shape, lambda i, nd=a.ndim: (0,) * nd))
        else:
            in_specs.append(
                pl.BlockSpec((_R, a.shape[1]), lambda i: (i, 0)))
    out_specs = [pl.BlockSpec((_R, d), lambda i: (i, 0)) for d in out_dims]
    out_shape = [jax.ShapeDtypeStruct((_N, d), jnp.float32) for d in out_dims]
    return pl.pallas_call(
        body, grid=(_G,), in_specs=in_specs,
        out_specs=out_specs, out_shape=out_shape)(*ins)


def _prep_body(st, nf, d0, d1, w1, ht_o, h_o, dinv_o):
    deg = d0[...] + d1[...] + 1.0
    dinv = jnp.where(deg > 0, lax.rsqrt(jnp.maximum(deg, 1e-12)), 0.0)
    h = (jnp.dot(nf[...], w1[0:5, :], preferred_element_type=jnp.float32)
         + st[...] * w1[5:6, :])
    h_o[...] = h
    ht_o[...] = dinv * h
    dinv_o[...] = dinv


def _d1_body(s1, h1, dinv_r, b, w, h_o, htlo_o, hthi_o):
    dinv = dinv_r[...]
    y = dinv * s1[...] + (dinv * dinv) * h1[...] + b[...]
    h = jnp.dot(y, w[...], preferred_element_type=jnp.float32)
    h_o[...] = h
    ht = dinv * h
    htlo_o[...] = ht[:, 0:32]
    hthi_o[...] = ht[:, 32:64]


def _d2_body(slo, shi, h2, dinv_r, b, w, h_o, htlo_o, hthi_o):
    dinv = dinv_r[...]
    h2v = h2[...]
    bv = b[...]
    ylo = dinv * slo[...] + (dinv * dinv) * h2v[:, 0:32] + bv[:, 0:32]
    yhi = dinv * shi[...] + (dinv * dinv) * h2v[:, 32:64] + bv[:, 32:64]
    h = (jnp.dot(ylo, w[0:32, :], preferred_element_type=jnp.float32)
         + jnp.dot(yhi, w[32:64, :], preferred_element_type=jnp.float32))
    h_o[...] = h
    ht = dinv * h
    htlo_o[...] = ht[:, 0:32]
    hthi_o[...] = ht[:, 32:64]


def _d3_body(slo, shi, h3, dinv_r, b, wp1, bp1, wp2, bp2, out):
    dinv = dinv_r[...]
    h3v = h3[...]
    bv = b[...]
    ylo = dinv * slo[...] + (dinv * dinv) * h3v[:, 0:32] + bv[:, 0:32]
    yhi = dinv * shi[...] + (dinv * dinv) * h3v[:, 32:64] + bv[:, 32:64]
    t = jnp.maximum(
        jnp.dot(ylo, wp1[0:32, :], preferred_element_type=jnp.float32)
        + jnp.dot(yhi, wp1[32:64, :], preferred_element_type=jnp.float32)
        + bp1[...], 0.0)
    z = jnp.dot(t, wp2[...], preferred_element_type=jnp.float32) + bp2[...]
    out[...] = jax.nn.sigmoid(z)


def kernel(states, env, node_features, edge_index, edge_attr,
           W1, b1, W2, b2, W3, b3, Wp1, bp1, Wp2, bp2):
    del env
    src = edge_index[0]
    dst = edge_index[1]
    pad = _EPAD - _E
    fill = (jnp.arange(pad, dtype=jnp.int32) * 797) % jnp.int32(_N)
    src_p = jnp.concatenate([src, fill]).reshape(_EPAD // 128, 128)
    dst_p = jnp.concatenate([dst, fill]).reshape(_EPAD // 128, 128)
    ew_flat = jnp.concatenate([edge_attr, jnp.zeros((pad,), jnp.float32)])
    ew_p = ew_flat.reshape(_EPAD // 128, 128)

    degp0, degp1 = _deg_call(dst_p, ew_p)
    d0 = degp0[:_N].reshape(_N, 1)
    d1 = degp1[:_N].reshape(_N, 1)

    def _unpad(o):
        return jnp.concatenate([o[:_HALF], o[_HALFP:_HALFP + _HALF]])

    st = states.reshape(_N, 1)
    ht1, h1, dinv = _row_call(
        _prep_body, (32, 32, 1),
        (st, node_features, d0, d1, W1),
        (False, False, False, False, True))

    edge32 = _make_edge_pass(1)
    edge64 = _make_edge_pass(2)

    (s1,) = edge32(src_p, dst_p, ew_flat, ht1)
    s1 = _unpad(s1)
    h2, ht2lo, ht2hi = _row_call(
        _d1_body, (64, 32, 32),
        (s1, h1, dinv, b1.reshape(1, 32), W2),
        (False, False, False, True, True))

    s2lo, s2hi = edge64(src_p, dst_p, ew_flat, ht2lo, ht2hi)
    s2lo, s2hi = _unpad(s2lo), _unpad(s2hi)
    h3, ht3lo, ht3hi = _row_call(
        _d2_body, (64, 32, 32),
        (s2lo, s2hi, h2, dinv, b2.reshape(1, 64), W3),
        (False, False, False, False, True, True))

    s3lo, s3hi = edge64(src_p, dst_p, ew_flat, ht3lo, ht3hi)
    s3lo, s3hi = _unpad(s3lo), _unpad(s3hi)
    (preds,) = _row_call(
        _d3_body, (1,),
        (s3lo, s3hi, h3, dinv, b3.reshape(1, 64),
         Wp1, bp1.reshape(1, 32), Wp2, bp2.reshape(1, 1)),
        (False, False, False, False, True, True, True, True, True))
    return preds.reshape(-1)


# speedup vs baseline: 1.6453x; 1.2666x over previous
import jax
import jax.numpy as jnp
from jax import lax
from jax.experimental import pallas as pl
from jax.experimental.pallas import tpu as pltpu
from jax.experimental.pallas import tpu_sc as plsc

_N = 100000
_E = 1600000
_NC, _NS = 2, 16
_HALF = _N // 2
_HALFP = 50176
_OWN = _HALFP // _NS
_ACC_ROWS = _HALFP
_SB = 1024
_CHK = 256
_EPAD = 1605632
_EPT = _EPAD // _NS
_NSB = _EPT // _SB
_KD = 256
_CHD = _KD // _CHK
_EPW = _EPAD // (_NC * _NS)
_NB_DEG = _EPW // _KD
_NPAD = 100352


def _mesh():
    return plsc.VectorSubcoreMesh(
        core_axis_name="c", subcore_axis_name="s",
        num_cores=_NC, num_subcores=_NS)


def _deg_body(dst_hbm, ew_hbm, out0_hbm, out1_hbm, idx_v, ew_v, zb_v, acc_sh):
    c = lax.axis_index("c")
    s = lax.axis_index("s")
    w = c * _NS + s

    def _z(i, _):
        zb_v[pl.ds(i * 16, 16)] = jnp.zeros((16,), jnp.float32)
        return 0
    lax.fori_loop(0, _KD // 16, _z, 0)

    npt = _NPAD // _NS
    for i in range(npt // _KD):
        pltpu.sync_copy(zb_v.at[pl.ds(0, _KD)],
                        acc_sh.at[pl.ds(s * npt + i * _KD, _KD)])
    rem = npt % _KD
    if rem:
        pltpu.sync_copy(zb_v.at[pl.ds(0, rem)],
                        acc_sh.at[pl.ds(s * npt + (npt // _KD) * _KD, rem)])
    plsc.subcore_barrier()

    row0 = w * (_EPW // _CHK)

    def _blk(b, _):
        rb = row0 + b * _CHD
        pltpu.sync_copy(dst_hbm.at[pl.ds(rb, _CHD)], idx_v)
        pltpu.sync_copy(ew_hbm.at[pl.ds(rb, _CHD)], ew_v)
        for ch in range(_CHD):
            pltpu.sync_copy(ew_v.at[ch], acc_sh.at[idx_v.at[ch]], add=True)
        return 0
    lax.fori_loop(0, _NB_DEG, _blk, 0)
    plsc.subcore_barrier()

    for out_hbm, cc in ((out0_hbm, 0), (out1_hbm, 1)):
        @pl.when(c == cc)
        def _():
            for i in range(npt // _KD):
                pltpu.sync_copy(acc_sh.at[pl.ds(s * npt + i * _KD, _KD)],
                                zb_v.at[pl.ds(0, _KD)])
                pltpu.sync_copy(zb_v.at[pl.ds(0, _KD)],
                                out_hbm.at[pl.ds(s * npt + i * _KD, _KD)])
            if rem:
                o = s * npt + (npt // _KD) * _KD
                pltpu.sync_copy(acc_sh.at[pl.ds(o, rem)],
                                zb_v.at[pl.ds(0, rem)])
                pltpu.sync_copy(zb_v.at[pl.ds(0, rem)],
                                out_hbm.at[pl.ds(o, rem)])


def _deg_call(dst_p, ew_p):
    fn = pl.kernel(
        _deg_body,
        out_type=(jax.ShapeDtypeStruct((_NPAD,), jnp.float32),
                  jax.ShapeDtypeStruct((_NPAD,), jnp.float32)),
        mesh=_mesh(),
        compiler_params=pltpu.CompilerParams(use_tc_tiling_on_sc=False),
        scratch_types=[
            pltpu.VMEM((_CHD, _CHK), jnp.int32),
            pltpu.VMEM((_CHD, _CHK), jnp.float32),
            pltpu.VMEM((_KD,), jnp.float32),
            pltpu.VMEM_SHARED((_NPAD,), jnp.float32),
        ],
    )
    return fn(dst_p, ew_p)


def _make_edge_pass(nf):
    def body(src_hbm, dst_hbm, ew_hbm, *rest):
        h_hbms = rest[:nf]
        out_hbms = rest[nf:2 * nf]
        (isA, idA, ewA, isB, idB, ewB, r0, r1, bnc, acc_sh,
         slA, slB, sg0, sg1, ss0, ss1) = rest[2 * nf:]
        stg = ((isA, idA, ewA, slA), (isB, idB, ewB, slB))
        rows = (r0, r1)
        sg = (sg0, sg1)
        ss = (ss0, ss1)
        c = lax.axis_index("c")
        s = lax.axis_index("s")
        base_node = c * _HALF
        row0 = s * (_EPT // _CHK)
        ebase = s * _EPT
        z16f = jnp.zeros((16,), jnp.float32)

        def _fire_lin(sb, st):
            rb = row0 + sb * (_SB // _CHK)
            pltpu.async_copy(src_hbm.at[pl.ds(rb, _SB // _CHK)], st[0], st[3])
            pltpu.async_copy(dst_hbm.at[pl.ds(rb, _SB // _CHK)], st[1], st[3])
            pltpu.async_copy(ew_hbm.at[pl.ds(ebase + sb * _SB, _SB)], st[2],
                             st[3])

        def _wait_lin(sb, st):
            rb = row0 + sb * (_SB // _CHK)
            pltpu.make_async_copy(src_hbm.at[pl.ds(rb, _SB // _CHK)], st[0],
                                  st[3]).wait()
            pltpu.make_async_copy(dst_hbm.at[pl.ds(rb, _SB // _CHK)], st[1],
                                  st[3]).wait()
            pltpu.make_async_copy(ew_hbm.at[pl.ds(ebase + sb * _SB, _SB)],
                                  st[2], st[3]).wait()

        def _fire_gather(h_hbm, st, ch, rset):
            pltpu.async_copy(h_hbm.at[st[0].at[ch]], rows[rset], sg[rset])

        def _wait_gather(h_hbm, st, ch, rset):
            pltpu.make_async_copy(h_hbm.at[st[0].at[ch]], rows[rset],
                                  sg[rset]).wait()

        def _fire_scatter(st, ch, rset):
            pltpu.async_copy(rows[rset], acc_sh.at[st[1].at[ch]],
                             ss[rset], add=True)

        def _wait_scatter(st, ch, rset):
            pltpu.make_async_copy(rows[rset], acc_sh.at[st[1].at[ch]],
                                  ss[rset]).wait()

        def _compute(st, ch, rset):
            id_v, ew_v, rows_v = st[1], st[2], rows[rset]

            def _ix(v, _):
                lb = v * 16
                d = id_v[ch, pl.ds(lb, 16)]
                e = ew_v[pl.ds(ch * _CHK + lb, 16)]
                loc = d - base_node
                ok = (loc >= 0) & (loc < _HALF)
                spread = jnp.bitwise_and(d, 16383)
                id_v[ch, pl.ds(lb, 16)] = jnp.where(ok, loc, spread)
                ew_v[pl.ds(ch * _CHK + lb, 16)] = jnp.where(ok, e, 0.0)
                return 0
            lax.fori_loop(0, _CHK // 16, _ix, 0)

            def _sc(g, _):
                e_vec = ew_v[pl.ds(ch * _CHK + g * 16, 16)]
                for l in range(16):
                    jj = g * 16 + l
                    e = e_vec[l]
                    rows_v[jj, pl.ds(0, 16)] = rows_v[jj, pl.ds(0, 16)] * e
                    rows_v[jj, pl.ds(16, 16)] = rows_v[jj, pl.ds(16, 16)] * e
                return 0
            lax.fori_loop(0, _CHK // 16, _sc, 0)

        for r in range(nf):
            def _zr(j, _):
                bnc[j, pl.ds(0, 16)] = z16f
                bnc[j, pl.ds(16, 16)] = z16f
                return 0
            lax.fori_loop(0, 128, _zr, 0)
            plsc.subcore_barrier()
            for i in range(_OWN // 128):
                pltpu.sync_copy(bnc,
                                acc_sh.at[pl.ds(s * _OWN + i * 128, 128)])
            arem = _OWN % 128
            if arem:
                pltpu.sync_copy(
                    bnc.at[pl.ds(0, arem)],
                    acc_sh.at[pl.ds(s * _OWN + (_OWN // 128) * 128, arem)])
            plsc.subcore_barrier()

            h_hbm = h_hbms[r]

            _fire_lin(0, stg[0])
            _fire_lin(1, stg[1])

            def _iter(i, _):
                _wait_lin(2 * i, stg[0])
                _wait_lin(2 * i + 1, stg[1])
                _fire_gather(h_hbm, stg[0], 0, 0)
                for t in range(8):
                    st = stg[0] if t < 4 else stg[1]
                    ch = t % 4
                    rset = t % 2
                    _wait_gather(h_hbm, st, ch, rset)
                    if t < 7:
                        nst = stg[0] if t + 1 < 4 else stg[1]
                        if t >= 1:
                            pst = stg[0] if t - 1 < 4 else stg[1]
                            _wait_scatter(pst, (t - 1) % 4, (t + 1) % 2)
                        _fire_gather(h_hbm, nst, (t + 1) % 4, (t + 1) % 2)
                    _compute(st, ch, rset)
                    _fire_scatter(st, ch, rset)
                    if t == 4:
                        @pl.when(i < _NSB // 2 - 1)
                        def _():
                            _fire_lin(2 * i + 2, stg[0])
                _wait_scatter(stg[1], 2, 0)
                _wait_scatter(stg[1], 3, 1)

                @pl.when(i < _NSB // 2 - 1)
                def _():
                    _fire_lin(2 * i + 3, stg[1])
                return 0
            lax.fori_loop(0, _NSB // 2, _iter, 0)
            plsc.subcore_barrier()

            out_hbm = out_hbms[r]
            off = 0
            for sz in (128,) * (_OWN // 128) + ((_OWN % 128,)
                                                if _OWN % 128 else ()):
                pltpu.sync_copy(acc_sh.at[pl.ds(s * _OWN + off, sz)],
                                bnc.at[pl.ds(0, sz)])
                pltpu.sync_copy(
                    bnc.at[pl.ds(0, sz)],
                    out_hbm.at[pl.ds(c * _HALFP + s * _OWN + off, sz)])
                off += sz

    out_type = tuple(
        jax.ShapeDtypeStruct((2 * _HALFP, 32), jnp.float32)
        for _ in range(nf))
    return pl.kernel(
        body,
        out_type=out_type,
        mesh=_mesh(),
        compiler_params=pltpu.CompilerParams(use_tc_tiling_on_sc=False),
        scratch_types=[
            pltpu.VMEM((_SB // _CHK, _CHK), jnp.int32),
            pltpu.VMEM((_SB // _CHK, _CHK), jnp.int32),
            pltpu.VMEM((_SB,), jnp.float32),
            pltpu.VMEM((_SB // _CHK, _CHK), jnp.int32),
            pltpu.VMEM((_SB // _CHK, _CHK), jnp.int32),
            pltpu.VMEM((_SB,), jnp.float32),
            pltpu.VMEM((_CHK, 32), jnp.float32),
            pltpu.VMEM((_CHK, 32), jnp.float32),
            pltpu.VMEM((128, 32), jnp.float32),
            pltpu.VMEM_SHARED((_ACC_ROWS, 32), jnp.float32),
            pltpu.SemaphoreType.DMA,
            pltpu.SemaphoreType.DMA,
            pltpu.SemaphoreType.DMA,
            pltpu.SemaphoreType.DMA,
            pltpu.SemaphoreType.DMA,
            pltpu.SemaphoreType.DMA,
        ],
    )


_R = 1000
_G = _N // _R


def _row_call(body, out_dims, ins, full_mask):
    in_specs = []
    for a, full in zip(ins, full_mask):
        if full:
            in_specs.append(
                pl.BlockSpec(a.shape, lambda i, nd=a.ndim: (0,) * nd))
        else:
            in_specs.append(
                pl.BlockSpec((_R, a.shape[1]), lambda i: (i, 0)))
    out_specs = [pl.BlockSpec((_R, d), lambda i: (i, 0)) for d in out_dims]
    out_shape = [jax.ShapeDtypeStruct((_N, d), jnp.float32) for d in out_dims]
    return pl.pallas_call(
        body, grid=(_G,), in_specs=in_specs,
        out_specs=out_specs, out_shape=out_shape)(*ins)


def _prep_body(st, nf, d0, d1, w1, ht_o, h_o, dinv_o):
    deg = d0[...] + d1[...] + 1.0
    dinv = jnp.where(deg > 0, lax.rsqrt(jnp.maximum(deg, 1e-12)), 0.0)
    h = (jnp.dot(nf[...], w1[0:5, :], preferred_element_type=jnp.float32)
         + st[...] * w1[5:6, :])
    h_o[...] = h
    ht_o[...] = dinv * h
    dinv_o[...] = dinv


def _d1_body(s1, h1, dinv_r, b, w, h_o, htlo_o, hthi_o):
    dinv = dinv_r[...]
    y = dinv * s1[...] + (dinv * dinv) * h1[...] + b[...]
    h = jnp.dot(y, w[...], preferred_element_type=jnp.float32)
    h_o[...] = h
    ht = dinv * h
    htlo_o[...] = ht[:, 0:32]
    hthi_o[...] = ht[:, 32:64]


def _d2_body(slo, shi, h2, dinv_r, b, w, h_o, htlo_o, hthi_o):
    dinv = dinv_r[...]
    h2v = h2[...]
    bv = b[...]
    ylo = dinv * slo[...] + (dinv * dinv) * h2v[:, 0:32] + bv[:, 0:32]
    yhi = dinv * shi[...] + (dinv * dinv) * h2v[:, 32:64] + bv[:, 32:64]
    h = (jnp.dot(ylo, w[0:32, :], preferred_element_type=jnp.float32)
         + jnp.dot(yhi, w[32:64, :], preferred_element_type=jnp.float32))
    h_o[...] = h
    ht = dinv * h
    htlo_o[...] = ht[:, 0:32]
    hthi_o[...] = ht[:, 32:64]


def _d3_body(slo, shi, h3, dinv_r, b, wp1, bp1, wp2, bp2, out):
    dinv = dinv_r[...]
    h3v = h3[...]
    bv = b[...]
    ylo = dinv * slo[...] + (dinv * dinv) * h3v[:, 0:32] + bv[:, 0:32]
    yhi = dinv * shi[...] + (dinv * dinv) * h3v[:, 32:64] + bv[:, 32:64]
    t = jnp.maximum(
        jnp.dot(ylo, wp1[0:32, :], preferred_element_type=jnp.float32)
        + jnp.dot(yhi, wp1[32:64, :], preferred_element_type=jnp.float32)
        + bp1[...], 0.0)
    z = jnp.dot(t, wp2[...], preferred_element_type=jnp.float32) + bp2[...]
    out[...] = jax.nn.sigmoid(z)


def kernel(states, env, node_features, edge_index, edge_attr,
           W1, b1, W2, b2, W3, b3, Wp1, bp1, Wp2, bp2):
    del env
    src = edge_index[0]
    dst = edge_index[1]
    pad = _EPAD - _E
    fill = (jnp.arange(pad, dtype=jnp.int32) * 797) % jnp.int32(_N)
    src_p = jnp.concatenate([src, fill]).reshape(_EPAD // _CHK, _CHK)
    dst_p = jnp.concatenate([dst, fill]).reshape(_EPAD // _CHK, _CHK)
    ew_flat = jnp.concatenate([edge_attr, jnp.zeros((pad,), jnp.float32)])
    ew_p = ew_flat.reshape(_EPAD // _CHK, _CHK)

    degp0, degp1 = _deg_call(dst_p, ew_p)
    d0 = degp0[:_N].reshape(_N, 1)
    d1 = degp1[:_N].reshape(_N, 1)

    def _unpad(o):
        return jnp.concatenate([o[:_HALF], o[_HALFP:_HALFP + _HALF]])

    st = states.reshape(_N, 1)
    ht1, h1, dinv = _row_call(
        _prep_body, (32, 32, 1),
        (st, node_features, d0, d1, W1),
        (False, False, False, False, True))

    edge32 = _make_edge_pass(1)
    edge64 = _make_edge_pass(2)

    (s1,) = edge32(src_p, dst_p, ew_flat, ht1)
    s1 = _unpad(s1)
    h2, ht2lo, ht2hi = _row_call(
        _d1_body, (64, 32, 32),
        (s1, h1, dinv, b1.reshape(1, 32), W2),
        (False, False, False, True, True))

    s2lo, s2hi = edge64(src_p, dst_p, ew_flat, ht2lo, ht2hi)
    s2lo, s2hi = _unpad(s2lo), _unpad(s2hi)
    h3, ht3lo, ht3hi = _row_call(
        _d2_body, (64, 32, 32),
        (s2lo, s2hi, h2, dinv, b2.reshape(1, 64), W3),
        (False, False, False, False, True, True))

    s3lo, s3hi = edge64(src_p, dst_p, ew_flat, ht3lo, ht3hi)
    s3lo, s3hi = _unpad(s3lo), _unpad(s3hi)
    (preds,) = _row_call(
        _d3_body, (1,),
        (s3lo, s3hi, h3, dinv, b3.reshape(1, 64),
         Wp1, bp1.reshape(1, 32), Wp2, bp2.reshape(1, 1)),
        (False, False, False, False, True, True, True, True, True))
    return preds.reshape(-1)
